# baseline (speedup 1.0000x reference)
import functools

import jax
import jax.numpy as jnp
from jax import lax
from jax.experimental import pallas as pl
from jax.experimental.pallas import tpu as pltpu
from jax.experimental.pallas import tpu_sc as plsc

N_NODES = 100000
K_VOL = 27
C_CH = 16

NW = 32
G = 448
CPK = (N_NODES + G - 1) // G
NUM_CHUNKS = K_VOL * CPK
LAST_NODE = N_NODES - G


def _chunk_coords(c):
    k = c // CPK
    node = jnp.minimum((c % CPK) * G, LAST_NODE)
    return k, pl.multiple_of(node, 8)


def _make_sc_gather():
    mesh = plsc.VectorSubcoreMesh(core_axis_name="c", subcore_axis_name="s")

    @functools.partial(
        pl.kernel,
        mesh=mesh,
        out_type=jax.ShapeDtypeStruct((K_VOL, C_CH, N_NODES), jnp.float32),
        scratch_types=[
            pltpu.VMEM((G,), jnp.int32),
            pltpu.VMEM((G,), jnp.int32),
            pltpu.VMEM((G, C_CH), jnp.float32),
            pltpu.VMEM((G, C_CH), jnp.float32),
            pltpu.VMEM((C_CH, G), jnp.float32),
            pltpu.VMEM((C_CH, G), jnp.float32),
            pltpu.VMEM_SHARED((N_NODES + 8, C_CH), jnp.float32),
            pltpu.SemaphoreType.DMA,
            pltpu.SemaphoreType.DMA,
            pltpu.SemaphoreType.DMA,
            pltpu.SemaphoreType.DMA,
        ],
        compiler_params=pltpu.CompilerParams(
            use_tc_tiling_on_sc=False, needs_layout_passes=False
        ),
    )
    def sc_gather(
        data_hbm, idxt_hbm, out_hbm,
        idx0, idx1, rows0, rows1, tr0, tr1, table_sp,
        sg0, sg1, sw0, sw1,
    ):
        idx = (idx0, idx1)
        rows = (rows0, rows1)
        tr = (tr0, tr1)
        sg = (sg0, sg1)
        sw = (sw0, sw1)

        wid = lax.axis_index("s") * 2 + lax.axis_index("c")
        n_mine = (NUM_CHUNKS + NW - 1 - wid) // NW
        lane = jax.lax.broadcasted_iota(jnp.int32, (16,), 0)

        @pl.when(lax.axis_index("s") == 0)
        def _():
            pltpu.sync_copy(data_hbm, table_sp)

        plsc.subcore_barrier()

        def stage_issue(t, b):
            c = wid + t * NW
            k, node = _chunk_coords(c)
            pltpu.sync_copy(idxt_hbm.at[k, pl.ds(node, G)], idx[b])

            @plsc.parallel_loop(0, G // 16, unroll=4)
            def _(i):
                sl = pl.ds(i * 16, 16)
                v = idx[b][sl]
                idx[b][sl] = jnp.where(v < 0, N_NODES, v)
            pltpu.async_copy(table_sp.at[idx[b]], rows[b], sg[b])

        def stage_retire(t, b):
            u = t - 1
            bu = b ^ 1
            w = t - 3

            @pl.when(w >= 0)
            def _():
                kw, nodew = _chunk_coords(wid + w * NW)
                pltpu.make_async_copy(
                    tr[bu], out_hbm.at[kw, :, pl.ds(nodew, G)], sw[bu]
                ).wait()

            pltpu.make_async_copy(
                table_sp.at[idx[bu]], rows[bu], sg[bu]
            ).wait()

            @plsc.parallel_loop(0, G // 16, unroll=8)
            def _(blk):
                row0 = blk * 16
                ridx = row0 + lane
                for ch in range(C_CH):
                    col = plsc.load_gather(
                        rows[bu], [ridx, jnp.full((16,), ch, jnp.int32)]
                    )
                    tr[bu][ch, pl.ds(row0, 16)] = col
            ku, nodeu = _chunk_coords(wid + u * NW)
            pltpu.async_copy(tr[bu], out_hbm.at[ku, :, pl.ds(nodeu, G)], sw[bu])

        def pair_body(tt, carry):
            for b in (0, 1):
                t = 2 * tt + b

                @pl.when(t < n_mine)
                def _():
                    stage_issue(t, b)

                @pl.when((t >= 1) & (t <= n_mine))
                def _():
                    stage_retire(t, b)

            return carry

        lax.fori_loop(0, (n_mine + 2) // 2, pair_body, 0)

        for b in (0, 1):
            wb = n_mine - 1 - ((n_mine - 1 + b) % 2)
            kb, nodeb = _chunk_coords(wid + wb * NW)
            pltpu.make_async_copy(
                tr[b], out_hbm.at[kb, :, pl.ds(nodeb, G)], sw[b]
            ).wait()

    return sc_gather


_sc_gather = _make_sc_gather()


@jax.jit
def kernel(data_in, octree):
    data_pad = jnp.concatenate(
        [data_in, jnp.zeros((8, C_CH), jnp.float32)], axis=0
    )
    out_t = _sc_gather(data_pad, octree.T)
    return out_t.transpose(2, 0, 1)


# speedup vs baseline: 1.1316x; 1.1316x over previous
import functools

import jax
import jax.numpy as jnp
from jax import lax
from jax.experimental import pallas as pl
from jax.experimental.pallas import tpu as pltpu
from jax.experimental.pallas import tpu_sc as plsc

N_NODES = 100000
K_VOL = 27
C_CH = 16

NW = 32
G = 448
CPK = (N_NODES + G - 1) // G
NUM_CHUNKS = K_VOL * CPK
LAST_NODE = N_NODES - G


def _chunk_coords(c):
    k = c // CPK
    node = jnp.minimum((c % CPK) * G, LAST_NODE)
    return k, pl.multiple_of(node, 8)


def _make_sc_gather():
    mesh = plsc.VectorSubcoreMesh(core_axis_name="c", subcore_axis_name="s")

    @functools.partial(
        pl.kernel,
        mesh=mesh,
        out_type=jax.ShapeDtypeStruct((K_VOL, C_CH, N_NODES), jnp.float32),
        scratch_types=[
            pltpu.VMEM((G,), jnp.int32),
            pltpu.VMEM((G,), jnp.int32),
            pltpu.VMEM((G, C_CH), jnp.float32),
            pltpu.VMEM((G, C_CH), jnp.float32),
            pltpu.VMEM((C_CH, G), jnp.float32),
            pltpu.VMEM((C_CH, G), jnp.float32),
            pltpu.VMEM_SHARED((N_NODES + 8, C_CH), jnp.float32),
            pltpu.SemaphoreType.DMA,
            pltpu.SemaphoreType.DMA,
            pltpu.SemaphoreType.DMA,
            pltpu.SemaphoreType.DMA,
        ],
        compiler_params=pltpu.CompilerParams(
            use_tc_tiling_on_sc=False, needs_layout_passes=False
        ),
    )
    def sc_gather(
        data_hbm, idxt_hbm, out_hbm,
        idx0, idx1, rows0, rows1, tr0, tr1, table_sp,
        sg0, sg1, sw0, sw1,
    ):
        idx = (idx0, idx1)
        rows = (rows0, rows1)
        tr = (tr0, tr1)
        sg = (sg0, sg1)
        sw = (sw0, sw1)

        wid = lax.axis_index("s") * 2 + lax.axis_index("c")
        n_mine = (NUM_CHUNKS + NW - 1 - wid) // NW
        lane = jax.lax.broadcasted_iota(jnp.int32, (16,), 0)

        @pl.when(lax.axis_index("s") == 0)
        def _():
            pltpu.sync_copy(data_hbm, table_sp)

        plsc.subcore_barrier()

        def stage_issue(t, b):
            c = wid + t * NW
            k, node = _chunk_coords(c)
            pltpu.sync_copy(idxt_hbm.at[k, pl.ds(node, G)], idx[b])

            @plsc.parallel_loop(0, G // 16, unroll=4)
            def _(i):
                sl = pl.ds(i * 16, 16)
                v = idx[b][sl]
                idx[b][sl] = jnp.where(v < 0, N_NODES, v)
            pltpu.async_copy(table_sp.at[idx[b]], rows[b], sg[b])

        def stage_retire(t, b):
            u = t - 1
            bu = b ^ 1
            w = t - 3

            @pl.when(w >= 0)
            def _():
                kw, nodew = _chunk_coords(wid + w * NW)
                pltpu.make_async_copy(
                    tr[bu], out_hbm.at[kw, :, pl.ds(nodew, G)], sw[bu]
                ).wait()

            pltpu.make_async_copy(
                table_sp.at[idx[bu]], rows[bu], sg[bu]
            ).wait()

            @plsc.parallel_loop(0, G // 16, unroll=4)
            def _(blk):
                row0 = blk * 16
                ridx = row0 + lane
                for ch in range(C_CH):
                    col = plsc.load_gather(
                        rows[bu], [ridx, jnp.full((16,), ch, jnp.int32)]
                    )
                    tr[bu][ch, pl.ds(row0, 16)] = col
            ku, nodeu = _chunk_coords(wid + u * NW)
            pltpu.async_copy(tr[bu], out_hbm.at[ku, :, pl.ds(nodeu, G)], sw[bu])

        def pair_body(tt, carry):
            for b in (0, 1):
                t = 2 * tt + b

                @pl.when(t < n_mine)
                def _():
                    stage_issue(t, b)

                @pl.when((t >= 1) & (t <= n_mine))
                def _():
                    stage_retire(t, b)

            return carry

        lax.fori_loop(0, (n_mine + 2) // 2, pair_body, 0)

        for b in (0, 1):
            wb = n_mine - 1 - ((n_mine - 1 + b) % 2)
            kb, nodeb = _chunk_coords(wid + wb * NW)
            pltpu.make_async_copy(
                tr[b], out_hbm.at[kb, :, pl.ds(nodeb, G)], sw[b]
            ).wait()

    return sc_gather


_sc_gather = _make_sc_gather()


@jax.jit
def kernel(data_in, octree):
    data_pad = jnp.concatenate(
        [data_in, jnp.zeros((8, C_CH), jnp.float32)], axis=0
    )
    out_t = _sc_gather(data_pad, octree.T)
    return out_t.transpose(2, 0, 1)


# speedup vs baseline: 1.3239x; 1.1700x over previous
import functools

import jax
import jax.numpy as jnp
from jax import lax
from jax.experimental import pallas as pl
from jax.experimental.pallas import tpu as pltpu
from jax.experimental.pallas import tpu_sc as plsc

N_NODES = 100000
K_VOL = 27
C_CH = 16

NW = 32
G = 448
CPK = (N_NODES + G - 1) // G
NUM_CHUNKS = K_VOL * CPK
LAST_NODE = N_NODES - G


def _chunk_coords(c):
    k = c // CPK
    node = jnp.minimum((c % CPK) * G, LAST_NODE)
    return k, pl.multiple_of(node, 8)


def _make_sc_gather():
    mesh = plsc.VectorSubcoreMesh(core_axis_name="c", subcore_axis_name="s")

    @functools.partial(
        pl.kernel,
        mesh=mesh,
        out_type=jax.ShapeDtypeStruct((K_VOL, C_CH, N_NODES), jnp.float32),
        scratch_types=[
            pltpu.VMEM((G,), jnp.int32),
            pltpu.VMEM((G,), jnp.int32),
            pltpu.VMEM((G, C_CH), jnp.float32),
            pltpu.VMEM((G, C_CH), jnp.float32),
            pltpu.VMEM((C_CH, G), jnp.float32),
            pltpu.VMEM((C_CH, G), jnp.float32),
            pltpu.VMEM_SHARED((N_NODES + 8, C_CH), jnp.float32),
            pltpu.SemaphoreType.DMA,
            pltpu.SemaphoreType.DMA,
            pltpu.SemaphoreType.DMA,
            pltpu.SemaphoreType.DMA,
            pltpu.SemaphoreType.DMA,
            pltpu.SemaphoreType.DMA,
        ],
        compiler_params=pltpu.CompilerParams(
            use_tc_tiling_on_sc=False, needs_layout_passes=False
        ),
    )
    def sc_gather(
        data_hbm, idxt_hbm, out_hbm,
        idx0, idx1, rows0, rows1, tr0, tr1, table_sp,
        sg0, sg1, sw0, sw1, si0, si1,
    ):
        idx = (idx0, idx1)
        rows = (rows0, rows1)
        tr = (tr0, tr1)
        sg = (sg0, sg1)
        sw = (sw0, sw1)
        si = (si0, si1)

        wid = lax.axis_index("s") * 2 + lax.axis_index("c")
        n_mine = (NUM_CHUNKS + NW - 1 - wid) // NW
        lane = jax.lax.broadcasted_iota(jnp.int32, (16,), 0)

        @pl.when(lax.axis_index("s") == 0)
        def _():
            pltpu.sync_copy(data_hbm, table_sp)

        plsc.subcore_barrier()

        def idx_load(t, b):
            k, node = _chunk_coords(wid + t * NW)
            pltpu.async_copy(idxt_hbm.at[k, pl.ds(node, G)], idx[b], si[b])

        def stage_wait_prev(t, b):
            bu = b ^ 1
            w = t - 3

            @pl.when(w >= 0)
            def _():
                kw, nodew = _chunk_coords(wid + w * NW)
                pltpu.make_async_copy(
                    tr[bu], out_hbm.at[kw, :, pl.ds(nodew, G)], sw[bu]
                ).wait()

            pltpu.make_async_copy(
                table_sp.at[idx[bu]], rows[bu], sg[bu]
            ).wait()

        def stage_issue(t, b):
            k, node = _chunk_coords(wid + t * NW)
            pltpu.make_async_copy(
                idxt_hbm.at[k, pl.ds(node, G)], idx[b], si[b]
            ).wait()

            def remap(i, carry2):
                sl = pl.ds(i * 16, 16)
                v = idx[b][sl]
                idx[b][sl] = jnp.where(v < 0, N_NODES, v)
                return carry2

            lax.fori_loop(0, G // 16, remap, 0, unroll=8)
            pltpu.async_copy(table_sp.at[idx[b]], rows[b], sg[b])

            @pl.when(t + 1 < n_mine)
            def _():
                idx_load(t + 1, b ^ 1)

        def stage_retire(t, b):
            u = t - 1
            bu = b ^ 1

            @plsc.parallel_loop(0, G // 16, unroll=4)
            def _(blk):
                row0 = blk * 16
                ridx = row0 + lane
                for ch in range(C_CH):
                    col = plsc.load_gather(
                        rows[bu], [ridx, jnp.full((16,), ch, jnp.int32)]
                    )
                    tr[bu][ch, pl.ds(row0, 16)] = col
            ku, nodeu = _chunk_coords(wid + u * NW)
            pltpu.async_copy(tr[bu], out_hbm.at[ku, :, pl.ds(nodeu, G)], sw[bu])

        def pair_body(tt, carry):
            for b in (0, 1):
                t = 2 * tt + b

                @pl.when((t >= 1) & (t <= n_mine))
                def _():
                    stage_wait_prev(t, b)

                @pl.when(t < n_mine)
                def _():
                    stage_issue(t, b)

                @pl.when((t >= 1) & (t <= n_mine))
                def _():
                    stage_retire(t, b)

            return carry

        idx_load(0, 0)
        lax.fori_loop(0, (n_mine + 2) // 2, pair_body, 0)

        for b in (0, 1):
            wb = n_mine - 1 - ((n_mine - 1 + b) % 2)
            kb, nodeb = _chunk_coords(wid + wb * NW)
            pltpu.make_async_copy(
                tr[b], out_hbm.at[kb, :, pl.ds(nodeb, G)], sw[b]
            ).wait()

    return sc_gather


_sc_gather = _make_sc_gather()


@jax.jit
def kernel(data_in, octree):
    data_pad = jnp.concatenate(
        [data_in, jnp.zeros((8, C_CH), jnp.float32)], axis=0
    )
    out_t = _sc_gather(data_pad, octree.T)
    return out_t.transpose(2, 0, 1)


# speedup vs baseline: 1.3945x; 1.0533x over previous
import functools

import jax
import jax.numpy as jnp
from jax import lax
from jax.experimental import pallas as pl
from jax.experimental.pallas import tpu as pltpu
from jax.experimental.pallas import tpu_sc as plsc

N_NODES = 100000
K_VOL = 27
C_CH = 16

NW = 32
G = 448
CPK = (N_NODES + G - 1) // G
NUM_CHUNKS = K_VOL * CPK
LAST_NODE = N_NODES - G


def _chunk_coords(c):
    k = c // CPK
    node = jnp.minimum((c % CPK) * G, LAST_NODE)
    return k, pl.multiple_of(node, 8)


def _make_sc_gather():
    mesh = plsc.VectorSubcoreMesh(core_axis_name="c", subcore_axis_name="s")

    @functools.partial(
        pl.kernel,
        mesh=mesh,
        out_type=jax.ShapeDtypeStruct((K_VOL, C_CH, N_NODES), jnp.float32),
        scratch_types=[
            pltpu.VMEM((G,), jnp.int32),
            pltpu.VMEM((G,), jnp.int32),
            pltpu.VMEM((G, C_CH), jnp.float32),
            pltpu.VMEM((G, C_CH), jnp.float32),
            pltpu.VMEM((C_CH, G), jnp.float32),
            pltpu.VMEM((C_CH, G), jnp.float32),
            pltpu.VMEM_SHARED((N_NODES + 8, C_CH), jnp.float32),
            pltpu.SemaphoreType.DMA,
            pltpu.SemaphoreType.DMA,
            pltpu.SemaphoreType.DMA,
            pltpu.SemaphoreType.DMA,
            pltpu.SemaphoreType.DMA,
            pltpu.SemaphoreType.DMA,
        ],
        compiler_params=pltpu.CompilerParams(
            use_tc_tiling_on_sc=False, needs_layout_passes=False
        ),
    )
    def sc_gather(
        data_hbm, idxt_hbm, out_hbm,
        idx0, idx1, rows0, rows1, tr0, tr1, table_sp,
        sg0, sg1, sw0, sw1, si0, si1,
    ):
        idx = (idx0, idx1)
        rows = (rows0, rows1)
        tr = (tr0, tr1)
        sg = (sg0, sg1)
        sw = (sw0, sw1)
        si = (si0, si1)

        wid = lax.axis_index("s") * 2 + lax.axis_index("c")
        n_mine = (NUM_CHUNKS + NW - 1 - wid) // NW
        lane = jax.lax.broadcasted_iota(jnp.int32, (16,), 0)

        @pl.when(lax.axis_index("s") == 0)
        def _():
            zero = jnp.zeros((16,), jnp.float32)
            for r in range(8):
                rows0[r, :] = zero
            pltpu.sync_copy(
                rows0.at[pl.ds(0, 8), :], table_sp.at[pl.ds(N_NODES, 8), :]
            )
            pltpu.sync_copy(data_hbm, table_sp.at[pl.ds(0, N_NODES), :])

        plsc.subcore_barrier()

        def idx_load(t, b):
            k, node = _chunk_coords(wid + t * NW)
            pltpu.async_copy(idxt_hbm.at[k, pl.ds(node, G)], idx[b], si[b])

        def stage_wait_prev(t, b):
            bu = b ^ 1
            w = t - 3

            @pl.when(w >= 0)
            def _():
                kw, nodew = _chunk_coords(wid + w * NW)
                pltpu.make_async_copy(
                    tr[bu], out_hbm.at[kw, :, pl.ds(nodew, G)], sw[bu]
                ).wait()

            pltpu.make_async_copy(
                table_sp.at[idx[bu]], rows[bu], sg[bu]
            ).wait()

        def stage_issue(t, b):
            k, node = _chunk_coords(wid + t * NW)
            pltpu.make_async_copy(
                idxt_hbm.at[k, pl.ds(node, G)], idx[b], si[b]
            ).wait()

            def remap(i, carry2):
                sl = pl.ds(i * 16, 16)
                v = idx[b][sl]
                idx[b][sl] = jnp.where(v < 0, N_NODES, v)
                return carry2

            lax.fori_loop(0, G // 16, remap, 0, unroll=8)
            pltpu.async_copy(table_sp.at[idx[b]], rows[b], sg[b])

            @pl.when(t + 1 < n_mine)
            def _():
                idx_load(t + 1, b ^ 1)

        def stage_retire(t, b):
            u = t - 1
            bu = b ^ 1

            @plsc.parallel_loop(0, G // 16, unroll=4)
            def _(blk):
                row0 = blk * 16
                ridx = row0 + lane
                for ch in range(C_CH):
                    col = plsc.load_gather(
                        rows[bu], [ridx, jnp.full((16,), ch, jnp.int32)]
                    )
                    tr[bu][ch, pl.ds(row0, 16)] = col
            ku, nodeu = _chunk_coords(wid + u * NW)
            pltpu.async_copy(tr[bu], out_hbm.at[ku, :, pl.ds(nodeu, G)], sw[bu])

        def pair_body(tt, carry):
            for b in (0, 1):
                t = 2 * tt + b

                @pl.when((t >= 1) & (t <= n_mine))
                def _():
                    stage_wait_prev(t, b)

                @pl.when(t < n_mine)
                def _():
                    stage_issue(t, b)

                @pl.when((t >= 1) & (t <= n_mine))
                def _():
                    stage_retire(t, b)

            return carry

        idx_load(0, 0)
        lax.fori_loop(0, (n_mine + 2) // 2, pair_body, 0)

        for b in (0, 1):
            wb = n_mine - 1 - ((n_mine - 1 + b) % 2)
            kb, nodeb = _chunk_coords(wid + wb * NW)
            pltpu.make_async_copy(
                tr[b], out_hbm.at[kb, :, pl.ds(nodeb, G)], sw[b]
            ).wait()

    return sc_gather


_sc_gather = _make_sc_gather()


@jax.jit
def kernel(data_in, octree):
    out_t = _sc_gather(data_in, octree.T)
    return out_t.transpose(2, 0, 1)
